# baseline (speedup 1.0000x reference)
import functools

import jax
import jax.numpy as jnp
from jax import lax
from jax.experimental import pallas as pl
from jax.experimental.pallas import tpu as pltpu
from jax.experimental.pallas import tpu_sc as plsc

N = 10000
E = 320000
D = 128
H = 20
C = 10

NPAD = 10112
WP = 32
DUMMY = 10016
NC = 2
NS = 16
NW = NC * NS
CK = 128
CHUNK = 512
NCHUNK = 40
SPLIT0 = 19
ROWS_PER_TILE = NPAD // NS


def _mm_body(x_ref, w_ref, o_ref):
    o_ref[...] = jnp.dot(x_ref[...], w_ref[...],
                         preferred_element_type=jnp.float32)


def _layer_body(p_ref, agg_ref, b_ref, w_ref, o_ref):
    h = p_ref[...] + agg_ref[0] + agg_ref[1] + b_ref[...]
    h = jnp.maximum(h, 0.0)
    row = lax.broadcasted_iota(jnp.int32, (NPAD, WP), 0)
    h = jnp.where(row < N, h, 0.0)
    o_ref[...] = jnp.dot(h, w_ref[...], preferred_element_type=jnp.float32)


def _final_body(p_ref, agg_ref, b_ref, wl_ref, bl_ref, o_ref):
    h = p_ref[...] + agg_ref[0] + agg_ref[1] + b_ref[...]
    h = jnp.maximum(h, 0.0)
    row = lax.broadcasted_iota(jnp.int32, (NPAD, WP), 0)
    h = jnp.where(row < N, h, 0.0)
    mx = jnp.max(h, axis=0, keepdims=True)
    mn = jnp.sum(h, axis=0, keepdims=True) / float(N)
    inp = jnp.concatenate([mx, mn], axis=1)
    o_ref[...] = jnp.dot(inp, wl_ref[...],
                         preferred_element_type=jnp.float32) + bl_ref[...]


_mm1 = pl.pallas_call(
    _mm_body, out_shape=jax.ShapeDtypeStruct((NPAD, WP), jnp.float32))

_layer = pl.pallas_call(
    _layer_body, out_shape=jax.ShapeDtypeStruct((NPAD, WP), jnp.float32))

_final = pl.pallas_call(
    _final_body, out_shape=jax.ShapeDtypeStruct((1, 128), jnp.float32))


def _sc_agg_body(p_hbm, src_hbm, dst_hbm, out_hbm,
                 src_v, dst_v, rows_v, acc_sh, tbl_sh, gsem):
    c = lax.axis_index("c")
    s = lax.axis_index("s")

    pltpu.sync_copy(src_hbm.at[s], src_v)
    pltpu.sync_copy(dst_hbm.at[s], dst_v)

    sl = pl.ds(s * ROWS_PER_TILE, ROWS_PER_TILE)
    pltpu.sync_copy(p_hbm.at[sl], tbl_sh.at[sl])

    def zrow(r, _):
        rows_v[0, r, pl.ds(0, 16)] = jnp.zeros((16,), jnp.float32)
        rows_v[0, r, pl.ds(WP - 16, 16)] = jnp.zeros((16,), jnp.float32)
        return 0
    lax.fori_loop(0, ROWS_PER_TILE, zrow, 0)
    pltpu.sync_copy(rows_v.at[0].at[pl.ds(0, ROWS_PER_TILE)],
                    acc_sh.at[sl])
    plsc.subcore_barrier()

    def run_chunks(ids):
        pltpu.async_copy(tbl_sh.at[src_v.at[ids[0]]], rows_v.at[0],
                         gsem.at[0])
        for j, m in enumerate(ids):
            b = j % 2
            pltpu.make_async_copy(tbl_sh.at[src_v.at[m]], rows_v.at[b],
                                  gsem.at[b]).wait()
            if j + 1 < len(ids):
                pltpu.async_copy(tbl_sh.at[src_v.at[ids[j + 1]]],
                                 rows_v.at[1 - b], gsem.at[1 - b])
            pltpu.sync_copy(rows_v.at[b], acc_sh.at[dst_v.at[m]], add=True)

    @pl.when(c == 0)
    def _():
        run_chunks(list(range(0, SPLIT0)))

    @pl.when(c == 1)
    def _():
        run_chunks(list(range(SPLIT0, NCHUNK)))
    plsc.subcore_barrier()

    pltpu.sync_copy(acc_sh.at[sl], out_hbm.at[c].at[sl])


_sc_agg = pl.kernel(
    _sc_agg_body,
    out_type=jax.ShapeDtypeStruct((NC, NPAD, WP), jnp.float32),
    mesh=plsc.VectorSubcoreMesh(core_axis_name="c", subcore_axis_name="s"),
    scratch_types=[
        pltpu.VMEM((NCHUNK, CHUNK), jnp.int32),
        pltpu.VMEM((NCHUNK, CHUNK), jnp.int32),
        pltpu.VMEM((2, CHUNK, WP), jnp.float32),
        pltpu.VMEM_SHARED((NPAD, WP), jnp.float32),
        pltpu.VMEM_SHARED((NPAD, WP), jnp.float32),
        pltpu.SemaphoreType.DMA((2,)),
    ],
    compiler_params=pltpu.CompilerParams(use_tc_tiling_on_sc=False),
)


def kernel(x, edge_index, W1, b1, W2, b2, W3, b3, Wl, bl):
    f32 = jnp.float32

    x_pad = jnp.zeros((NPAD, D), f32).at[:N].set(x)
    W1p = jnp.zeros((D, WP), f32).at[:, :H].set(W1)
    W2p = jnp.zeros((WP, WP), f32).at[:H, :H].set(W2)
    W3p = jnp.zeros((WP, WP), f32).at[:H, :H].set(W3)
    b1p = jnp.zeros((1, WP), f32).at[0, :H].set(b1)
    b2p = jnp.zeros((1, WP), f32).at[0, :H].set(b2)
    b3p = jnp.zeros((1, WP), f32).at[0, :H].set(b3)
    Wlp = (jnp.zeros((2 * WP, 128), f32)
           .at[:H, :C].set(Wl[:H])
           .at[WP:WP + H, :C].set(Wl[H:]))
    blp = jnp.zeros((1, 128), f32).at[0, :C].set(bl)

    EP = NS * NCHUNK * CHUNK
    srcp = jnp.full((EP,), DUMMY, jnp.int32).at[:E].set(
        edge_index[0]).reshape(NS, NCHUNK, CHUNK)
    dfill = (N + (jnp.arange(EP, dtype=jnp.int32) % (NPAD - N)))
    dstp = dfill.at[:E].set(edge_index[1]).reshape(NS, NCHUNK, CHUNK)

    p1 = _mm1(x_pad, W1p)
    a1 = _sc_agg(p1, srcp, dstp)
    p2 = _layer(p1, a1, b1p, W2p)
    a2 = _sc_agg(p2, srcp, dstp)
    p3 = _layer(p2, a2, b2p, W3p)
    a3 = _sc_agg(p3, srcp, dstp)
    out = _final(p3, a3, b3p, Wlp, blp)
    return out[:, :C]


# speedup vs baseline: 1.0181x; 1.0181x over previous
import functools

import jax
import jax.numpy as jnp
from jax import lax
from jax.experimental import pallas as pl
from jax.experimental.pallas import tpu as pltpu
from jax.experimental.pallas import tpu_sc as plsc

N = 10000
E = 320000
D = 128
H = 20
C = 10

NPAD = 10112
WP = 32
DUMMY = 10016
NC = 2
NS = 16
NW = NC * NS
CK = 128
CHUNK = 512
NCHUNK = 40
SPLIT0 = 20
ROWS_PER_TILE = NPAD // NS


def _mm_body(x_ref, w_ref, o_ref):
    o_ref[...] = jnp.dot(x_ref[...], w_ref[...],
                         preferred_element_type=jnp.float32)


def _layer_body(p_ref, agg_ref, b_ref, w_ref, o_ref):
    h = p_ref[...] + agg_ref[0] + agg_ref[1] + b_ref[...]
    h = jnp.maximum(h, 0.0)
    row = lax.broadcasted_iota(jnp.int32, (NPAD, WP), 0)
    h = jnp.where(row < N, h, 0.0)
    o_ref[...] = jnp.dot(h, w_ref[...], preferred_element_type=jnp.float32)


def _final_body(p_ref, agg_ref, b_ref, wl_ref, bl_ref, o_ref):
    h = p_ref[...] + agg_ref[0] + agg_ref[1] + b_ref[...]
    h = jnp.maximum(h, 0.0)
    row = lax.broadcasted_iota(jnp.int32, (NPAD, WP), 0)
    h = jnp.where(row < N, h, 0.0)
    mx = jnp.max(h, axis=0, keepdims=True)
    mn = jnp.sum(h, axis=0, keepdims=True) / float(N)
    inp = jnp.concatenate([mx, mn], axis=1)
    o_ref[...] = jnp.dot(inp, wl_ref[...],
                         preferred_element_type=jnp.float32) + bl_ref[...]


_mm1 = pl.pallas_call(
    _mm_body, out_shape=jax.ShapeDtypeStruct((NPAD, WP), jnp.float32))

_layer = pl.pallas_call(
    _layer_body, out_shape=jax.ShapeDtypeStruct((NPAD, WP), jnp.float32))

_final = pl.pallas_call(
    _final_body, out_shape=jax.ShapeDtypeStruct((1, 128), jnp.float32))


def _sc_agg_body(p_hbm, src_hbm, dst_hbm, out_hbm,
                 src_v, dst_v, rows_v, acc_sh, tbl_sh, gsem):
    c = lax.axis_index("c")
    s = lax.axis_index("s")

    pltpu.sync_copy(src_hbm.at[s], src_v)
    pltpu.sync_copy(dst_hbm.at[s], dst_v)

    sl = pl.ds(s * ROWS_PER_TILE, ROWS_PER_TILE)
    pltpu.sync_copy(p_hbm.at[sl], tbl_sh.at[sl])

    def zrow(r, _):
        rows_v[0, r, pl.ds(0, 16)] = jnp.zeros((16,), jnp.float32)
        rows_v[0, r, pl.ds(WP - 16, 16)] = jnp.zeros((16,), jnp.float32)
        return 0
    lax.fori_loop(0, ROWS_PER_TILE, zrow, 0)
    pltpu.sync_copy(rows_v.at[0].at[pl.ds(0, ROWS_PER_TILE)],
                    acc_sh.at[sl])
    plsc.subcore_barrier()

    def run_chunks(ids):
        pltpu.async_copy(tbl_sh.at[src_v.at[ids[0]]], rows_v.at[0],
                         gsem.at[0])
        for j, m in enumerate(ids):
            b = j % 2
            pltpu.make_async_copy(tbl_sh.at[src_v.at[m]], rows_v.at[b],
                                  gsem.at[b]).wait()
            if j + 1 < len(ids):
                pltpu.async_copy(tbl_sh.at[src_v.at[ids[j + 1]]],
                                 rows_v.at[1 - b], gsem.at[1 - b])
            pltpu.sync_copy(rows_v.at[b], acc_sh.at[dst_v.at[m]], add=True)

    @pl.when(c == 0)
    def _():
        run_chunks(list(range(0, SPLIT0)))

    @pl.when(c == 1)
    def _():
        run_chunks(list(range(SPLIT0, NCHUNK)))
    plsc.subcore_barrier()

    pltpu.sync_copy(acc_sh.at[sl], out_hbm.at[c].at[sl])


_sc_agg = pl.kernel(
    _sc_agg_body,
    out_type=jax.ShapeDtypeStruct((NC, NPAD, WP), jnp.float32),
    mesh=plsc.VectorSubcoreMesh(core_axis_name="c", subcore_axis_name="s"),
    scratch_types=[
        pltpu.VMEM((NCHUNK, CHUNK), jnp.int32),
        pltpu.VMEM((NCHUNK, CHUNK), jnp.int32),
        pltpu.VMEM((2, CHUNK, WP), jnp.float32),
        pltpu.VMEM_SHARED((NPAD, WP), jnp.float32),
        pltpu.VMEM_SHARED((NPAD, WP), jnp.float32),
        pltpu.SemaphoreType.DMA((2,)),
    ],
    compiler_params=pltpu.CompilerParams(use_tc_tiling_on_sc=False),
)


def kernel(x, edge_index, W1, b1, W2, b2, W3, b3, Wl, bl):
    f32 = jnp.float32

    x_pad = jnp.zeros((NPAD, D), f32).at[:N].set(x)
    W1p = jnp.zeros((D, WP), f32).at[:, :H].set(W1)
    W2p = jnp.zeros((WP, WP), f32).at[:H, :H].set(W2)
    W3p = jnp.zeros((WP, WP), f32).at[:H, :H].set(W3)
    b1p = jnp.zeros((1, WP), f32).at[0, :H].set(b1)
    b2p = jnp.zeros((1, WP), f32).at[0, :H].set(b2)
    b3p = jnp.zeros((1, WP), f32).at[0, :H].set(b3)
    Wlp = (jnp.zeros((2 * WP, 128), f32)
           .at[:H, :C].set(Wl[:H])
           .at[WP:WP + H, :C].set(Wl[H:]))
    blp = jnp.zeros((1, 128), f32).at[0, :C].set(bl)

    EP = NS * NCHUNK * CHUNK
    srcp = jnp.full((EP,), DUMMY, jnp.int32).at[:E].set(
        edge_index[0]).reshape(NS, NCHUNK, CHUNK)
    dfill = (N + (jnp.arange(EP, dtype=jnp.int32) % (NPAD - N)))
    dstp = dfill.at[:E].set(edge_index[1]).reshape(NS, NCHUNK, CHUNK)

    p1 = _mm1(x_pad, W1p)
    a1 = _sc_agg(p1, srcp, dstp)
    p2 = _layer(p1, a1, b1p, W2p)
    a2 = _sc_agg(p2, srcp, dstp)
    p3 = _layer(p2, a2, b2p, W3p)
    a3 = _sc_agg(p3, srcp, dstp)
    out = _final(p3, a3, b3p, Wlp, blp)
    return out[:, :C]


# speedup vs baseline: 1.0840x; 1.0647x over previous
import functools

import jax
import jax.numpy as jnp
from jax import lax
from jax.experimental import pallas as pl
from jax.experimental.pallas import tpu as pltpu
from jax.experimental.pallas import tpu_sc as plsc

N = 10000
E = 320000
D = 128
H = 20
C = 10

NPAD = 10112
WP = 32
DUMMY = 10016
NC = 2
NS = 16
NW = NC * NS
CK = 128
CHUNK = 512
NCHUNK = 40
SPLIT0 = 20
ROWS_PER_TILE = NPAD // NS


def _mm_body(x_ref, w_ref, o_ref):
    o_ref[...] = jnp.dot(x_ref[...], w_ref[...],
                         preferred_element_type=jnp.float32)


def _layer_body(p_ref, agg_ref, b_ref, w_ref, o_ref):
    h = p_ref[...] + agg_ref[0] + agg_ref[1] + b_ref[...]
    h = jnp.maximum(h, 0.0)
    row = lax.broadcasted_iota(jnp.int32, (NPAD, WP), 0)
    h = jnp.where(row < N, h, 0.0)
    o_ref[...] = jnp.dot(h, w_ref[...], preferred_element_type=jnp.float32)


def _final_body(p_ref, agg_ref, b_ref, wl_ref, bl_ref, o_ref):
    h = p_ref[...] + agg_ref[0] + agg_ref[1] + b_ref[...]
    h = jnp.maximum(h, 0.0)
    row = lax.broadcasted_iota(jnp.int32, (NPAD, WP), 0)
    h = jnp.where(row < N, h, 0.0)
    mx = jnp.max(h, axis=0, keepdims=True)
    mn = jnp.sum(h, axis=0, keepdims=True) / float(N)
    inp = jnp.concatenate([mx, mn], axis=1)
    o_ref[...] = jnp.dot(inp, wl_ref[...],
                         preferred_element_type=jnp.float32) + bl_ref[...]


_mm1 = pl.pallas_call(
    _mm_body, out_shape=jax.ShapeDtypeStruct((NPAD, WP), jnp.float32))

_layer = pl.pallas_call(
    _layer_body, out_shape=jax.ShapeDtypeStruct((NPAD, WP), jnp.float32))

_final = pl.pallas_call(
    _final_body, out_shape=jax.ShapeDtypeStruct((1, 128), jnp.float32))


def _sc_agg_body(p_hbm, src_hbm, dst_hbm, out_hbm,
                 src_v, dst_v, rows_v, acc_sh, tbl_sh, gsem):
    c = lax.axis_index("c")
    s = lax.axis_index("s")

    sl = pl.ds(s * ROWS_PER_TILE, ROWS_PER_TILE)
    pltpu.async_copy(src_hbm.at[s], src_v, gsem.at[0])
    pltpu.async_copy(dst_hbm.at[s], dst_v, gsem.at[1])
    pltpu.async_copy(p_hbm.at[sl], tbl_sh.at[sl], gsem.at[2])

    def zrow(r, _):
        rows_v[0, r, pl.ds(0, 16)] = jnp.zeros((16,), jnp.float32)
        rows_v[0, r, pl.ds(WP - 16, 16)] = jnp.zeros((16,), jnp.float32)
        return 0
    lax.fori_loop(0, ROWS_PER_TILE, zrow, 0)
    pltpu.sync_copy(rows_v.at[0].at[pl.ds(0, ROWS_PER_TILE)],
                    acc_sh.at[sl])
    pltpu.make_async_copy(src_hbm.at[s], src_v, gsem.at[0]).wait()
    pltpu.make_async_copy(dst_hbm.at[s], dst_v, gsem.at[1]).wait()
    pltpu.make_async_copy(p_hbm.at[sl], tbl_sh.at[sl], gsem.at[2]).wait()
    plsc.subcore_barrier()

    def run_chunks(ids):
        pltpu.async_copy(tbl_sh.at[src_v.at[ids[0]]], rows_v.at[0],
                         gsem.at[0])
        for j, m in enumerate(ids):
            b = j % 2
            pltpu.make_async_copy(tbl_sh.at[src_v.at[m]], rows_v.at[b],
                                  gsem.at[b]).wait()
            if j + 1 < len(ids):
                pltpu.async_copy(tbl_sh.at[src_v.at[ids[j + 1]]],
                                 rows_v.at[1 - b], gsem.at[1 - b])
            pltpu.sync_copy(rows_v.at[b], acc_sh.at[dst_v.at[m]], add=True)

    @pl.when(c == 0)
    def _():
        run_chunks(list(range(0, SPLIT0)))

    @pl.when(c == 1)
    def _():
        run_chunks(list(range(SPLIT0, NCHUNK)))
    plsc.subcore_barrier()

    pltpu.sync_copy(acc_sh.at[sl], out_hbm.at[c].at[sl])


_sc_agg = pl.kernel(
    _sc_agg_body,
    out_type=jax.ShapeDtypeStruct((NC, NPAD, WP), jnp.float32),
    mesh=plsc.VectorSubcoreMesh(core_axis_name="c", subcore_axis_name="s"),
    scratch_types=[
        pltpu.VMEM((NCHUNK, CHUNK), jnp.int32),
        pltpu.VMEM((NCHUNK, CHUNK), jnp.int32),
        pltpu.VMEM((2, CHUNK, WP), jnp.float32),
        pltpu.VMEM_SHARED((NPAD, WP), jnp.float32),
        pltpu.VMEM_SHARED((NPAD, WP), jnp.float32),
        pltpu.SemaphoreType.DMA((3,)),
    ],
    compiler_params=pltpu.CompilerParams(use_tc_tiling_on_sc=False),
)


def kernel(x, edge_index, W1, b1, W2, b2, W3, b3, Wl, bl):
    f32 = jnp.float32

    x_pad = jnp.zeros((NPAD, D), f32).at[:N].set(x)
    W1p = jnp.zeros((D, WP), f32).at[:, :H].set(W1)
    W2p = jnp.zeros((WP, WP), f32).at[:H, :H].set(W2)
    W3p = jnp.zeros((WP, WP), f32).at[:H, :H].set(W3)
    b1p = jnp.zeros((1, WP), f32).at[0, :H].set(b1)
    b2p = jnp.zeros((1, WP), f32).at[0, :H].set(b2)
    b3p = jnp.zeros((1, WP), f32).at[0, :H].set(b3)
    Wlp = (jnp.zeros((2 * WP, 128), f32)
           .at[:H, :C].set(Wl[:H])
           .at[WP:WP + H, :C].set(Wl[H:]))
    blp = jnp.zeros((1, 128), f32).at[0, :C].set(bl)

    EP = NS * NCHUNK * CHUNK
    srcp = jnp.full((EP,), DUMMY, jnp.int32).at[:E].set(
        edge_index[0]).reshape(NS, NCHUNK, CHUNK)
    dfill = (N + (jnp.arange(EP, dtype=jnp.int32) % (NPAD - N)))
    dstp = dfill.at[:E].set(edge_index[1]).reshape(NS, NCHUNK, CHUNK)

    p1 = _mm1(x_pad, W1p)
    a1 = _sc_agg(p1, srcp, dstp)
    p2 = _layer(p1, a1, b1p, W2p)
    a2 = _sc_agg(p2, srcp, dstp)
    p3 = _layer(p2, a2, b2p, W3p)
    a3 = _sc_agg(p3, srcp, dstp)
    out = _final(p3, a3, b3p, Wlp, blp)
    return out[:, :C]


# speedup vs baseline: 1.1082x; 1.0224x over previous
import functools

import jax
import jax.numpy as jnp
from jax import lax
from jax.experimental import pallas as pl
from jax.experimental.pallas import tpu as pltpu
from jax.experimental.pallas import tpu_sc as plsc

N = 10000
E = 320000
D = 128
H = 20
C = 10

NPAD = 10112
WP = 32
DUMMY = 10016
NC = 2
NS = 16
NW = NC * NS
CK = 128
CHUNK = 1024
NCHUNK = 20
HALF = NCHUNK // 2
ROWS_PER_TILE = NPAD // NS


def _mm_body(x_ref, w_ref, o_ref):
    o_ref[...] = jnp.dot(x_ref[...], w_ref[...],
                         preferred_element_type=jnp.float32)


def _layer_body(p_ref, agg_ref, b_ref, w_ref, o_ref):
    h = p_ref[...] + agg_ref[0] + agg_ref[1] + b_ref[...]
    h = jnp.maximum(h, 0.0)
    row = lax.broadcasted_iota(jnp.int32, (NPAD, WP), 0)
    h = jnp.where(row < N, h, 0.0)
    o_ref[...] = jnp.dot(h, w_ref[...], preferred_element_type=jnp.float32)


def _final_body(p_ref, agg_ref, b_ref, wl_ref, bl_ref, o_ref):
    h = p_ref[...] + agg_ref[0] + agg_ref[1] + b_ref[...]
    h = jnp.maximum(h, 0.0)
    row = lax.broadcasted_iota(jnp.int32, (NPAD, WP), 0)
    h = jnp.where(row < N, h, 0.0)
    mx = jnp.max(h, axis=0, keepdims=True)
    mn = jnp.sum(h, axis=0, keepdims=True) / float(N)
    inp = jnp.concatenate([mx, mn], axis=1)
    o_ref[...] = jnp.dot(inp, wl_ref[...],
                         preferred_element_type=jnp.float32) + bl_ref[...]


_mm1 = pl.pallas_call(
    _mm_body, out_shape=jax.ShapeDtypeStruct((NPAD, WP), jnp.float32))

_layer = pl.pallas_call(
    _layer_body, out_shape=jax.ShapeDtypeStruct((NPAD, WP), jnp.float32))

_final = pl.pallas_call(
    _final_body, out_shape=jax.ShapeDtypeStruct((1, 128), jnp.float32))


def _sc_agg_body(p_hbm, src_hbm, dst_hbm, out_hbm,
                 src_v, dst_v, rows_v, acc_sh, tbl_sh, gsem):
    c = lax.axis_index("c")
    s = lax.axis_index("s")

    sl = pl.ds(s * ROWS_PER_TILE, ROWS_PER_TILE)
    hl = pl.ds(c * HALF, HALF)
    pltpu.async_copy(src_hbm.at[s].at[hl], src_v, gsem.at[0])
    pltpu.async_copy(dst_hbm.at[s].at[hl], dst_v, gsem.at[1])
    pltpu.async_copy(p_hbm.at[sl], tbl_sh.at[sl], gsem.at[2])

    def zrow(r, _):
        rows_v[0, r, pl.ds(0, 16)] = jnp.zeros((16,), jnp.float32)
        rows_v[0, r, pl.ds(WP - 16, 16)] = jnp.zeros((16,), jnp.float32)
        return 0
    lax.fori_loop(0, ROWS_PER_TILE, zrow, 0)
    pltpu.sync_copy(rows_v.at[0].at[pl.ds(0, ROWS_PER_TILE)],
                    acc_sh.at[sl])
    pltpu.make_async_copy(src_hbm.at[s].at[hl], src_v, gsem.at[0]).wait()
    pltpu.make_async_copy(dst_hbm.at[s].at[hl], dst_v, gsem.at[1]).wait()
    pltpu.make_async_copy(p_hbm.at[sl], tbl_sh.at[sl], gsem.at[2]).wait()
    plsc.subcore_barrier()

    def run_chunks(ids):
        pltpu.async_copy(tbl_sh.at[src_v.at[ids[0]]], rows_v.at[0],
                         gsem.at[0])
        for j, m in enumerate(ids):
            b = j % 2
            pltpu.make_async_copy(tbl_sh.at[src_v.at[m]], rows_v.at[b],
                                  gsem.at[b]).wait()
            if j + 1 < len(ids):
                pltpu.async_copy(tbl_sh.at[src_v.at[ids[j + 1]]],
                                 rows_v.at[1 - b], gsem.at[1 - b])
            pltpu.sync_copy(rows_v.at[b], acc_sh.at[dst_v.at[m]], add=True)

    run_chunks(list(range(HALF)))
    plsc.subcore_barrier()

    pltpu.sync_copy(acc_sh.at[sl], out_hbm.at[c].at[sl])


_sc_agg = pl.kernel(
    _sc_agg_body,
    out_type=jax.ShapeDtypeStruct((NC, NPAD, WP), jnp.float32),
    mesh=plsc.VectorSubcoreMesh(core_axis_name="c", subcore_axis_name="s"),
    scratch_types=[
        pltpu.VMEM((HALF, CHUNK), jnp.int32),
        pltpu.VMEM((HALF, CHUNK), jnp.int32),
        pltpu.VMEM((2, CHUNK, WP), jnp.float32),
        pltpu.VMEM_SHARED((NPAD, WP), jnp.float32),
        pltpu.VMEM_SHARED((NPAD, WP), jnp.float32),
        pltpu.SemaphoreType.DMA((3,)),
    ],
    compiler_params=pltpu.CompilerParams(use_tc_tiling_on_sc=False),
)


def kernel(x, edge_index, W1, b1, W2, b2, W3, b3, Wl, bl):
    f32 = jnp.float32

    x_pad = jnp.zeros((NPAD, D), f32).at[:N].set(x)
    W1p = jnp.zeros((D, WP), f32).at[:, :H].set(W1)
    W2p = jnp.zeros((WP, WP), f32).at[:H, :H].set(W2)
    W3p = jnp.zeros((WP, WP), f32).at[:H, :H].set(W3)
    b1p = jnp.zeros((1, WP), f32).at[0, :H].set(b1)
    b2p = jnp.zeros((1, WP), f32).at[0, :H].set(b2)
    b3p = jnp.zeros((1, WP), f32).at[0, :H].set(b3)
    Wlp = (jnp.zeros((2 * WP, 128), f32)
           .at[:H, :C].set(Wl[:H])
           .at[WP:WP + H, :C].set(Wl[H:]))
    blp = jnp.zeros((1, 128), f32).at[0, :C].set(bl)

    EP = NS * NCHUNK * CHUNK
    srcp = jnp.full((EP,), DUMMY, jnp.int32).at[:E].set(
        edge_index[0]).reshape(NS, NCHUNK, CHUNK)
    dfill = (N + (jnp.arange(EP, dtype=jnp.int32) % (NPAD - N)))
    dstp = dfill.at[:E].set(edge_index[1]).reshape(NS, NCHUNK, CHUNK)

    p1 = _mm1(x_pad, W1p)
    a1 = _sc_agg(p1, srcp, dstp)
    p2 = _layer(p1, a1, b1p, W2p)
    a2 = _sc_agg(p2, srcp, dstp)
    p3 = _layer(p2, a2, b2p, W3p)
    a3 = _sc_agg(p3, srcp, dstp)
    out = _final(p3, a3, b3p, Wlp, blp)
    return out[:, :C]
